# baseline (speedup 1.0000x reference)
import functools

import jax
import jax.numpy as jnp
from jax import lax
from jax.experimental import pallas as pl
from jax.experimental.pallas import tpu as pltpu
from jax.experimental.pallas import tpu_sc as plsc

V = 100000
D = 128
B = 16384
NC = 2
NS = 16
NW = NC * NS
BPW = B // NW
NCHUNK = 4
CH = BPW // NCHUNK
NCH = D // 16

_mesh = plsc.VectorSubcoreMesh(core_axis_name="c", subcore_axis_name="s")


def _bcast(ref, lane):
    idx = jnp.full((16,), lane, jnp.int32)
    return ref[...].at[idx].get(mode="promise_in_bounds")


@functools.partial(
    pl.kernel,
    out_type=jax.ShapeDtypeStruct((B,), jnp.float32),
    mesh=_mesh,
    scratch_types=[
        pltpu.VMEM((BPW,), jnp.int32),
        [pltpu.VMEM((CH, D), jnp.float32) for _ in range(NCHUNK)],
        pltpu.VMEM((D,), jnp.float32),
        pltpu.VMEM((2 * BPW,), jnp.float32),
        pltpu.VMEM((16,), jnp.float32),
        pltpu.VMEM((16,), jnp.float32),
        pltpu.VMEM((16,), jnp.float32),
        pltpu.VMEM((BPW,), jnp.float32),
        [pltpu.SemaphoreType.DMA for _ in range(NCHUNK)],
    ],
)
def _sc_fused(table_hbm, idx_hbm, w1_hbm, feats_hbm, b1_hbm, wfc_hbm,
              bfc_hbm, out_hbm, idx_v, rows_v, w1_v, fi_v, sb1_v, swfc_v,
              sbfc_v, out_v, sems):
    wid = lax.axis_index("s") * NC + lax.axis_index("c")
    base = wid * BPW

    pltpu.sync_copy(idx_hbm.at[pl.ds(base, BPW)], idx_v)
    cps = [
        pltpu.async_copy(
            table_hbm.at[idx_v.at[pl.ds(k * CH, CH)]], rows_v[k], sems[k])
        for k in range(NCHUNK)
    ]
    pltpu.sync_copy(w1_hbm, w1_v)
    pltpu.sync_copy(feats_hbm.at[pl.ds(2 * base, 2 * BPW)], fi_v)
    pltpu.sync_copy(b1_hbm, sb1_v.at[pl.ds(0, 1)])
    pltpu.sync_copy(wfc_hbm, swfc_v.at[pl.ds(0, 3)])
    pltpu.sync_copy(bfc_hbm, sbfc_v.at[pl.ds(0, 1)])

    w1c = [w1_v[pl.ds(16 * c, 16)] for c in range(NCH)]
    c_b1 = _bcast(sb1_v, 0)
    c_w0 = _bcast(swfc_v, 0)
    c_w1 = _bcast(swfc_v, 1)
    c_w2 = _bcast(swfc_v, 2)
    c_bfc = _bcast(sbfc_v, 0)
    lanes = lax.iota(jnp.int32, 16)
    one = jnp.full((16,), 1.0, jnp.float32)
    two = jnp.full((16,), 2.0, jnp.float32)
    lo8 = lanes < 8
    p_even = (lanes * 2) % 16
    p_odd = p_even + 1
    p8 = lanes ^ 8
    p4 = lanes ^ 4
    p2 = lanes ^ 2
    p1 = lanes ^ 1

    def perm(v, p):
        return v.at[p].get(mode="promise_in_bounds")

    def hsum(v):
        v = v + perm(v, p8)
        v = v + perm(v, p4)
        v = v + perm(v, p2)
        v = v + perm(v, p1)
        return v

    def do_chunk(rows_ref, off):
        @plsc.parallel_loop(0, CH // 16, 1, unroll=1)
        def _(g):
            y = jnp.zeros((16,), jnp.float32)
            for j in range(16):
                r = 16 * g + j
                ea = rows_ref[r, pl.ds(0, 16)] * w1c[0]
                eb = rows_ref[r, pl.ds(16, 16)] * w1c[1]
                for c in range(2, NCH, 2):
                    ea = ea + rows_ref[r, pl.ds(16 * c, 16)] * w1c[c]
                    eb = eb + rows_ref[r, pl.ds(16 * (c + 1), 16)] * w1c[c + 1]
                y = jnp.where(lanes == j, hsum(ea + eb), y)
            x = y + c_b1
            ax = jnp.abs(x)
            e = jnp.exp(two * ax)
            t = one - two / (e + one)
            t = jnp.where(x < 0.0, -t, t)
            b16 = off + 16 * g
            va = fi_v[pl.ds(2 * b16, 16)]
            vb = fi_v[pl.ds(2 * b16 + 16, 16)]
            f0 = jnp.where(lo8, perm(va, p_even), perm(vb, p_even))
            f1 = jnp.where(lo8, perm(va, p_odd), perm(vb, p_odd))
            out_v[pl.ds(b16, 16)] = f0 * c_w0 + f1 * c_w1 + t * c_w2 + c_bfc

    for k in range(NCHUNK):
        cps[k].wait()
        do_chunk(rows_v[k], k * CH)

    pltpu.sync_copy(out_v, out_hbm.at[pl.ds(base, BPW)])


def kernel(adj, feats, table, W1, b1, Wfc, bfc):
    idx = adj.astype(jnp.int32)
    ret = _sc_fused(table, idx, W1.reshape(D), feats.reshape(2 * B),
                    b1, Wfc.reshape(3), bfc)
    return ret.reshape(B, 1)


# speedup vs baseline: 1.0105x; 1.0105x over previous
import functools

import jax
import jax.numpy as jnp
from jax import lax
from jax.experimental import pallas as pl
from jax.experimental.pallas import tpu as pltpu
from jax.experimental.pallas import tpu_sc as plsc

V = 100000
D = 128
B = 16384
NC = 2
NS = 16
NW = NC * NS
BPW = B // NW
HALF = BPW // 2
NCH = D // 16

_OFF_FEATS = B
_OFF_W = 3 * B

_mesh = plsc.VectorSubcoreMesh(core_axis_name="c", subcore_axis_name="s")


@functools.partial(
    pl.kernel,
    out_type=jax.ShapeDtypeStruct((B,), jnp.float32),
    mesh=_mesh,
    scratch_types=[
        pltpu.VMEM((BPW,), jnp.float32),
        pltpu.VMEM((BPW,), jnp.int32),
        pltpu.VMEM((HALF, D), jnp.float32),
        pltpu.VMEM((HALF, D), jnp.float32),
        pltpu.VMEM((2 * BPW,), jnp.float32),
        pltpu.VMEM((D + 16,), jnp.float32),
        pltpu.VMEM((BPW,), jnp.float32),
        pltpu.SemaphoreType.DMA,
        pltpu.SemaphoreType.DMA,
    ],
)
def _sc_fused(table_hbm, packed_hbm, out_hbm,
              idxf_v, idx_v, rows0_v, rows1_v, fi_v, w_v, out_v, sem0, sem1):
    wid = lax.axis_index("s") * NC + lax.axis_index("c")
    base = wid * BPW

    pltpu.sync_copy(packed_hbm.at[pl.ds(base, BPW)], idxf_v)

    @plsc.parallel_loop(0, BPW // 16, 1, unroll=4)
    def _(i):
        idx_v[pl.ds(16 * i, 16)] = idxf_v[pl.ds(16 * i, 16)].astype(jnp.int32)
    cp0 = pltpu.async_copy(table_hbm.at[idx_v.at[pl.ds(0, HALF)]], rows0_v,
                           sem0)
    cp1 = pltpu.async_copy(table_hbm.at[idx_v.at[pl.ds(HALF, HALF)]], rows1_v,
                           sem1)
    pltpu.sync_copy(packed_hbm.at[pl.ds(_OFF_FEATS + 2 * base, 2 * BPW)],
                    fi_v)
    pltpu.sync_copy(packed_hbm.at[pl.ds(_OFF_W, D + 16)], w_v)

    w1c = [w_v[pl.ds(16 * c, 16)] for c in range(NCH)]
    sm = w_v[pl.ds(D, 16)]

    lanes = lax.iota(jnp.int32, 16)

    def perm(v, p):
        return v.at[p].get(mode="promise_in_bounds")

    def bcast(v, lane):
        return perm(v, jnp.full((16,), lane, jnp.int32))

    c_b1 = bcast(sm, 0)
    c_w0 = bcast(sm, 1)
    c_w1 = bcast(sm, 2)
    c_w2 = bcast(sm, 3)
    c_bfc = bcast(sm, 4)
    one = jnp.full((16,), 1.0, jnp.float32)
    two = jnp.full((16,), 2.0, jnp.float32)
    lo8 = lanes < 8
    p_even = (lanes * 2) % 16
    p_odd = p_even + 1
    p8 = lanes ^ 8
    p4 = lanes ^ 4
    p2 = lanes ^ 2
    p1 = lanes ^ 1

    def hsum(v):
        v = v + perm(v, p8)
        v = v + perm(v, p4)
        v = v + perm(v, p2)
        v = v + perm(v, p1)
        return v

    def do_half(rows_ref, off):
        @plsc.parallel_loop(0, HALF // 16, 1, unroll=1)
        def _(g):
            y = jnp.zeros((16,), jnp.float32)
            for j in range(16):
                r = 16 * g + j
                ea = rows_ref[r, pl.ds(0, 16)] * w1c[0]
                eb = rows_ref[r, pl.ds(16, 16)] * w1c[1]
                for c in range(2, NCH, 2):
                    ea = ea + rows_ref[r, pl.ds(16 * c, 16)] * w1c[c]
                    eb = eb + rows_ref[r, pl.ds(16 * (c + 1), 16)] * w1c[c + 1]
                y = jnp.where(lanes == j, hsum(ea + eb), y)
            x = y + c_b1
            ax = jnp.abs(x)
            e = jnp.exp(two * ax)
            t = one - two / (e + one)
            t = jnp.where(x < 0.0, -t, t)
            b16 = off + 16 * g
            va = fi_v[pl.ds(2 * b16, 16)]
            vb = fi_v[pl.ds(2 * b16 + 16, 16)]
            f0 = jnp.where(lo8, perm(va, p_even), perm(vb, p_even))
            f1 = jnp.where(lo8, perm(va, p_odd), perm(vb, p_odd))
            out_v[pl.ds(b16, 16)] = f0 * c_w0 + f1 * c_w1 + t * c_w2 + c_bfc

    cp0.wait()
    do_half(rows0_v, 0)
    cp1.wait()
    do_half(rows1_v, HALF)

    pltpu.sync_copy(out_v, out_hbm.at[pl.ds(base, BPW)])


def kernel(adj, feats, table, W1, b1, Wfc, bfc):
    sm = jnp.concatenate([b1, Wfc.reshape(3), bfc,
                          jnp.zeros((11,), jnp.float32)])
    packed = jnp.concatenate([adj.astype(jnp.float32), feats.reshape(2 * B),
                              W1.reshape(D), sm])
    ret = _sc_fused(table, packed)
    return ret.reshape(B, 1)


# speedup vs baseline: 1.0243x; 1.0137x over previous
import functools

import jax
import jax.numpy as jnp
from jax import lax
from jax.experimental import pallas as pl
from jax.experimental.pallas import tpu as pltpu
from jax.experimental.pallas import tpu_sc as plsc

V = 100000
D = 128
B = 16384
NC = 2
NS = 16
NW = NC * NS
BPW = B // NW
HALF = BPW // 2
NCH = D // 16

_mesh = plsc.VectorSubcoreMesh(core_axis_name="c", subcore_axis_name="s")


@functools.partial(
    pl.kernel,
    out_type=jax.ShapeDtypeStruct((B,), jnp.float32),
    mesh=_mesh,
    scratch_types=[
        pltpu.VMEM((BPW,), jnp.int32),
        pltpu.VMEM((HALF, D), jnp.float32),
        pltpu.VMEM((HALF, D), jnp.float32),
        pltpu.VMEM((2 * BPW,), jnp.float32),
        pltpu.VMEM((D + 16,), jnp.float32),
        pltpu.VMEM((BPW,), jnp.float32),
        pltpu.SemaphoreType.DMA,
        pltpu.SemaphoreType.DMA,
        pltpu.SemaphoreType.DMA,
        pltpu.SemaphoreType.DMA,
    ],
)
def _sc_fused(table_hbm, idx_hbm, feats_hbm, wc_hbm, out_hbm,
              idx_v, rows0_v, rows1_v, fi_v, wc_v, out_v,
              sem0, sem1, sem2, sem3):
    wid = lax.axis_index("s") * NC + lax.axis_index("c")
    base = wid * BPW

    pltpu.sync_copy(idx_hbm.at[pl.ds(base, BPW)], idx_v)
    cp0 = pltpu.async_copy(table_hbm.at[idx_v.at[pl.ds(0, HALF)]], rows0_v,
                           sem0)
    cp1 = pltpu.async_copy(table_hbm.at[idx_v.at[pl.ds(HALF, HALF)]], rows1_v,
                           sem1)
    cpf = pltpu.async_copy(feats_hbm.at[pl.ds(2 * base, 2 * BPW)], fi_v, sem2)
    cpw = pltpu.async_copy(wc_hbm, wc_v, sem3)

    cpw.wait()
    w1c = [wc_v[pl.ds(16 * c, 16)] for c in range(NCH)]
    sm = wc_v[pl.ds(D, 16)]

    lanes = lax.iota(jnp.int32, 16)

    def perm(v, p):
        return v.at[p].get(mode="promise_in_bounds")

    def bcast(v, lane):
        return perm(v, jnp.full((16,), lane, jnp.int32))

    c_b1 = bcast(sm, 0)
    c_w0 = bcast(sm, 1)
    c_w1 = bcast(sm, 2)
    c_w2 = bcast(sm, 3)
    c_bfc = bcast(sm, 4)
    one = jnp.full((16,), 1.0, jnp.float32)
    two = jnp.full((16,), 2.0, jnp.float32)
    lo8 = lanes < 8
    p_even = (lanes * 2) % 16
    p_odd = p_even + 1
    stages = [(jnp.bitwise_and(lanes, d) == 0, lanes ^ d) for d in (8, 4, 2, 1)]

    def merge(a, b, m, pd):
        return jnp.where(m, a, perm(b, pd)) + jnp.where(m, perm(a, pd), b)

    cpf.wait()

    def do_half(rows_ref, off):
        @plsc.parallel_loop(0, HALF // 16, 1, unroll=2)
        def _(g):
            vs = []
            for j in range(16):
                r = 16 * g + j
                ea = rows_ref[r, pl.ds(0, 16)] * w1c[0]
                eb = rows_ref[r, pl.ds(16, 16)] * w1c[1]
                for c in range(2, NCH, 2):
                    ea = ea + rows_ref[r, pl.ds(16 * c, 16)] * w1c[c]
                    eb = eb + rows_ref[r, pl.ds(16 * (c + 1), 16)] * w1c[c + 1]
                vs.append(ea + eb)
            for m, pd in stages:
                n = len(vs) // 2
                vs = [merge(vs[i], vs[i + n], m, pd) for i in range(n)]
            x = vs[0] + c_b1
            ax = jnp.abs(x)
            e = jnp.exp(two * ax)
            t = one - two / (e + one)
            t = jnp.where(x < 0.0, -t, t)
            b16 = off + 16 * g
            va = fi_v[pl.ds(2 * b16, 16)]
            vb = fi_v[pl.ds(2 * b16 + 16, 16)]
            f0 = jnp.where(lo8, perm(va, p_even), perm(vb, p_even))
            f1 = jnp.where(lo8, perm(va, p_odd), perm(vb, p_odd))
            out_v[pl.ds(b16, 16)] = f0 * c_w0 + f1 * c_w1 + t * c_w2 + c_bfc

    cp0.wait()
    do_half(rows0_v, 0)
    cp1.wait()
    do_half(rows1_v, HALF)

    pltpu.sync_copy(out_v, out_hbm.at[pl.ds(base, BPW)])


def kernel(adj, feats, table, W1, b1, Wfc, bfc):
    wc = jnp.concatenate([W1.reshape(D), b1, Wfc.reshape(3), bfc,
                          jnp.zeros((11,), jnp.float32)])
    ret = _sc_fused(table, adj.astype(jnp.int32), feats.reshape(2 * B), wc)
    return ret.reshape(B, 1)


# speedup vs baseline: 1.3968x; 1.3636x over previous
import functools

import jax
import jax.numpy as jnp
from jax import lax
from jax.experimental import pallas as pl
from jax.experimental.pallas import tpu as pltpu
from jax.experimental.pallas import tpu_sc as plsc

V = 100000
D = 128
B = 16384
NC = 2
NS = 16
NW = NC * NS
BPW = B // NW
HALF = BPW // 2
NCH = D // 16

_mesh = plsc.VectorSubcoreMesh(core_axis_name="c", subcore_axis_name="s")


@functools.partial(
    pl.kernel,
    out_type=jax.ShapeDtypeStruct((B,), jnp.float32),
    mesh=_mesh,
    scratch_types=[
        pltpu.VMEM((BPW,), jnp.int32),
        pltpu.VMEM((HALF, D), jnp.float32),
        pltpu.VMEM((HALF, D), jnp.float32),
        pltpu.VMEM((BPW,), jnp.float32),
        pltpu.VMEM((BPW,), jnp.float32),
        pltpu.VMEM((D + 16,), jnp.float32),
        pltpu.VMEM((BPW,), jnp.float32),
        pltpu.SemaphoreType.DMA,
        pltpu.SemaphoreType.DMA,
        pltpu.SemaphoreType.DMA,
        pltpu.SemaphoreType.DMA,
        pltpu.SemaphoreType.DMA,
    ],
)
def _sc_fused(table_hbm, idx_hbm, feats_hbm, wc_hbm, out_hbm,
              idx_v, rows0_v, rows1_v, f0_v, f1_v, wc_v, out_v,
              sem0, sem1, sem2, sem3, sem4):
    wid = lax.axis_index("s") * NC + lax.axis_index("c")
    base = wid * BPW

    pltpu.sync_copy(idx_hbm.at[pl.ds(base, BPW)], idx_v)
    cp0 = pltpu.async_copy(table_hbm.at[idx_v.at[pl.ds(0, HALF)]], rows0_v,
                           sem0)
    cp1 = pltpu.async_copy(table_hbm.at[idx_v.at[pl.ds(HALF, HALF)]], rows1_v,
                           sem1)
    cpf = pltpu.async_copy(feats_hbm.at[pl.ds(base, BPW)], f0_v, sem2)
    cpg = pltpu.async_copy(feats_hbm.at[pl.ds(B + base, BPW)], f1_v, sem3)
    cpw = pltpu.async_copy(wc_hbm, wc_v, sem4)

    cpw.wait()
    w1c = [wc_v[pl.ds(16 * c, 16)] for c in range(NCH)]
    sm = wc_v[pl.ds(D, 16)]

    lanes = lax.iota(jnp.int32, 16)

    def perm(v, p):
        return v.at[p].get(mode="promise_in_bounds")

    def bcast(v, lane):
        return perm(v, jnp.full((16,), lane, jnp.int32))

    c_b1 = bcast(sm, 0)
    c_w0 = bcast(sm, 1)
    c_w1 = bcast(sm, 2)
    c_w2 = bcast(sm, 3)
    c_bfc = bcast(sm, 4)
    one = jnp.full((16,), 1.0, jnp.float32)
    two = jnp.full((16,), 2.0, jnp.float32)
    lo8 = lanes < 8
    p_even = (lanes * 2) % 16
    p_odd = p_even + 1
    stages = [(jnp.bitwise_and(lanes, d) == 0, lanes ^ d) for d in (8, 4, 2, 1)]

    def merge(a, b, m, pd):
        return jnp.where(m, a, perm(b, pd)) + jnp.where(m, perm(a, pd), b)

    cpf.wait()
    cpg.wait()

    def do_half(rows_ref, off):
        @plsc.parallel_loop(0, HALF // 16, 1, unroll=2)
        def _(g):
            vs = []
            for j in range(16):
                r = 16 * g + j
                ea = rows_ref[r, pl.ds(0, 16)] * w1c[0]
                eb = rows_ref[r, pl.ds(16, 16)] * w1c[1]
                for c in range(2, NCH, 2):
                    ea = ea + rows_ref[r, pl.ds(16 * c, 16)] * w1c[c]
                    eb = eb + rows_ref[r, pl.ds(16 * (c + 1), 16)] * w1c[c + 1]
                vs.append(ea + eb)
            for m, pd in stages:
                n = len(vs) // 2
                vs = [merge(vs[i], vs[i + n], m, pd) for i in range(n)]
            x = vs[0] + c_b1
            ax = jnp.abs(x)
            e = jnp.exp(two * ax)
            t = one - two / (e + one)
            t = jnp.where(x < 0.0, -t, t)
            b16 = off + 16 * g
            f0 = f0_v[pl.ds(b16, 16)]
            f1 = f1_v[pl.ds(b16, 16)]
            out_v[pl.ds(b16, 16)] = f0 * c_w0 + f1 * c_w1 + t * c_w2 + c_bfc

    cp0.wait()
    do_half(rows0_v, 0)
    cp1.wait()
    do_half(rows1_v, HALF)

    pltpu.sync_copy(out_v, out_hbm.at[pl.ds(base, BPW)])


def kernel(adj, feats, table, W1, b1, Wfc, bfc):
    wc = jnp.concatenate([W1.reshape(D), b1, Wfc.reshape(3), bfc,
                          jnp.zeros((11,), jnp.float32)])
    ret = _sc_fused(table, adj.astype(jnp.int32), feats.T.reshape(2 * B), wc)
    return ret.reshape(B, 1)
